# baseline (speedup 1.0000x reference)
import functools

import jax
import jax.numpy as jnp
from jax import lax
from jax.experimental import pallas as pl
from jax.experimental.pallas import tpu as pltpu
from jax.experimental.pallas import tpu_sc as plsc

EMBED_DIM = 32
BATCH = 16384
FIELDS = 26

NC = 2
NS = 16
NW = NC * NS

ROWS_PER_W = BATCH // NW
CHUNK_ROWS = 64
N_CHUNKS = ROWS_PER_W // CHUNK_ROWS


def _build():
    mesh = plsc.VectorSubcoreMesh(
        core_axis_name="c", subcore_axis_name="s", num_cores=NC, num_subcores=NS
    )

    @functools.partial(
        pl.kernel,
        mesh=mesh,
        out_type=jax.ShapeDtypeStruct((BATCH, FIELDS, EMBED_DIM), jnp.float32),
        scratch_types=[
            pltpu.VMEM((CHUNK_ROWS, FIELDS), jnp.int32),
            pltpu.VMEM((CHUNK_ROWS, FIELDS), jnp.int32),
            pltpu.VMEM((CHUNK_ROWS, FIELDS, EMBED_DIM), jnp.float32),
            pltpu.VMEM((CHUNK_ROWS, FIELDS, EMBED_DIM), jnp.float32),
            pltpu.SemaphoreType.DMA,
            pltpu.SemaphoreType.DMA,
            pltpu.SemaphoreType.DMA,
            pltpu.SemaphoreType.DMA,
        ],
        compiler_params=pltpu.CompilerParams(use_tc_tiling_on_sc=False),
    )
    def gather_kernel(idx_hbm, table_hbm, out_hbm, i0, i1, r0, r1, gs0, gs1, os0, os1):
        ibuf, rbuf = [i0, i1], [r0, r1]
        gsem, osem = [gs0, gs1], [os0, os1]
        wid = lax.axis_index("s") * NC + lax.axis_index("c")
        base = wid * ROWS_PER_W

        def off(c):
            return base + c * CHUNK_ROWS

        def load_idx(c):
            pltpu.sync_copy(idx_hbm.at[pl.ds(off(c), CHUNK_ROWS)], ibuf[c % 2])

        def fire_gathers(c):
            p = c % 2

            def body(r, _):
                pltpu.async_copy(table_hbm.at[ibuf[p].at[r]], rbuf[p].at[r], gsem[p])
                return 0

            lax.fori_loop(0, CHUNK_ROWS, body, 0)

        def drain_gathers(c):
            p = c % 2

            def body(r, _):
                pltpu.make_async_copy(
                    table_hbm.at[ibuf[p].at[0]], rbuf[p].at[0], gsem[p]
                ).wait()
                return 0

            lax.fori_loop(0, CHUNK_ROWS, body, 0)

        def store(c):
            p = c % 2
            return pltpu.async_copy(
                rbuf[p], out_hbm.at[pl.ds(off(c), CHUNK_ROWS)], osem[p]
            )

        load_idx(0)
        fire_gathers(0)
        load_idx(1)
        outs = [None, None]
        for c in range(N_CHUNKS):
            p, q = c % 2, (c + 1) % 2
            drain_gathers(c)
            if c >= 1:
                outs[q].wait()
            outs[p] = store(c)
            if c + 1 < N_CHUNKS:
                fire_gathers(c + 1)
            if c + 2 < N_CHUNKS:
                load_idx(c + 2)
        outs[(N_CHUNKS - 1) % 2].wait()

    return gather_kernel


_gather = _build()


@jax.jit
def kernel(data, table):
    return _gather(data.astype(jnp.int32), table)
